# baseline (device time: 97806 ns/iter reference)
import jax
import jax.numpy as jnp
from jax import lax
from jax.experimental import pallas as pl
from jax.experimental.pallas import tpu as pltpu

N_DEV = 4
N_TOK = 1024
D_MODEL = 512
D_FF = 1024
E_PER = 4
CHUNK = N_TOK // N_DEV


def kernel(x, router_W, route_idx, expert_W, shared_W):
    def body(x_ref, rw_ref, idx_ref, ew_ref, sw_ref, out_ref,
             acc_ref, comm_ref, send_sems, recv_sems):
        my_i = lax.axis_index("i")
        left = (my_i - 1) % N_DEV
        right = (my_i + 1) % N_DEV

        barrier_sem = pltpu.get_barrier_semaphore()
        for nbr in [left, right]:
            pl.semaphore_signal(
                barrier_sem, inc=1,
                device_id=(nbr,), device_id_type=pl.DeviceIdType.MESH,
            )
        pl.semaphore_wait(barrier_sem, 2)

        x_val = x_ref[:, :]
        scores = jnp.dot(x_val, rw_ref[:, :], preferred_element_type=jnp.float32)
        s_max = jnp.max(scores, axis=-1, keepdims=True)
        p = jnp.exp(scores - s_max)
        probs = p / jnp.sum(p, axis=-1, keepdims=True)

        route = idx_ref[:, :]
        n_experts = probs.shape[-1]
        eids = lax.broadcasted_iota(jnp.int32, (N_TOK, n_experts), 1)

        acc = jnp.zeros((N_TOK, D_FF), jnp.float32)
        for le in range(E_PER):
            e = my_i * E_PER + le
            prob_e = jnp.sum(
                jnp.where(eids == e, probs, 0.0), axis=-1, keepdims=True
            )
            gate = jnp.where(route == e, prob_e, 0.0)
            acc = acc + jnp.dot(
                x_val * gate, ew_ref[le], preferred_element_type=jnp.float32
            )
        acc_ref[:, :] = acc

        for s in range(N_DEV - 1):
            c_send = (my_i - s) % N_DEV
            rdma = pltpu.make_async_remote_copy(
                src_ref=acc_ref.at[pl.ds(c_send * CHUNK, CHUNK), :],
                dst_ref=comm_ref.at[s],
                send_sem=send_sems.at[0, s],
                recv_sem=recv_sems.at[0, s],
                device_id=(right,),
                device_id_type=pl.DeviceIdType.MESH,
            )
            rdma.start()
            rdma.wait()
            c_recv = (my_i - 1 - s) % N_DEV
            acc_ref[pl.ds(c_recv * CHUNK, CHUNK), :] = (
                acc_ref[pl.ds(c_recv * CHUNK, CHUNK), :] + comm_ref[s]
            )

        c_own = (my_i + 1) % N_DEV
        out_ref[pl.ds(c_own * CHUNK, CHUNK), :] = (
            acc_ref[pl.ds(c_own * CHUNK, CHUNK), :]
            + jnp.dot(
                x_ref[pl.ds(c_own * CHUNK, CHUNK), :],
                sw_ref[:, :],
                preferred_element_type=jnp.float32,
            )
        )

        for s in range(N_DEV - 1):
            c_send = (my_i + 1 - s) % N_DEV
            rdma = pltpu.make_async_remote_copy(
                src_ref=out_ref.at[pl.ds(c_send * CHUNK, CHUNK), :],
                dst_ref=out_ref.at[pl.ds(c_send * CHUNK, CHUNK), :],
                send_sem=send_sems.at[1, s],
                recv_sem=recv_sems.at[1, s],
                device_id=(right,),
                device_id_type=pl.DeviceIdType.MESH,
            )
            rdma.start()
            rdma.wait()

    return pl.pallas_call(
        body,
        out_shape=jax.ShapeDtypeStruct((N_TOK, D_FF), jnp.float32),
        in_specs=[
            pl.BlockSpec(memory_space=pltpu.VMEM),
            pl.BlockSpec(memory_space=pltpu.VMEM),
            pl.BlockSpec(memory_space=pltpu.VMEM),
            pl.BlockSpec(memory_space=pltpu.VMEM),
            pl.BlockSpec(memory_space=pltpu.VMEM),
        ],
        out_specs=pl.BlockSpec(memory_space=pltpu.VMEM),
        scratch_shapes=[
            pltpu.VMEM((N_TOK, D_FF), jnp.float32),
            pltpu.VMEM((N_DEV - 1, CHUNK, D_FF), jnp.float32),
            pltpu.SemaphoreType.DMA((2, N_DEV - 1)),
            pltpu.SemaphoreType.DMA((2, N_DEV - 1)),
        ],
        compiler_params=pltpu.CompilerParams(collective_id=0),
    )(x, router_W, route_idx, expert_W, shared_W)


# device time: 47817 ns/iter; 2.0454x vs baseline; 2.0454x over previous
import jax
import jax.numpy as jnp
from jax import lax
from jax.experimental import pallas as pl
from jax.experimental.pallas import tpu as pltpu

N_DEV = 4
N_TOK = 1024
D_MODEL = 512
D_FF = 1024
E_PER = 4
CHUNK = N_TOK // N_DEV
HALF = CHUNK // 2


def kernel(x, router_W, route_idx, expert_W, shared_W):
    def body(x_ref, rw_ref, idx_ref, ew_ref, sw_ref, out_ref,
             acc_ref, ewb_ref, rs_send, rs_recv, ag_send0, ag_recv,
             rs_ssem, rs_rsem, ag_ssem, ag_rsem):
        my_i = lax.axis_index("i")
        left = (my_i - 1) % N_DEV
        right = (my_i + 1) % N_DEV
        nbr = [right, left]

        barrier_sem = pltpu.get_barrier_semaphore()
        for n in [left, right]:
            pl.semaphore_signal(
                barrier_sem, inc=1,
                device_id=(n,), device_id_type=pl.DeviceIdType.MESH,
            )
        pl.semaphore_wait(barrier_sem, 2)

        x_val = x_ref[:, :]
        scores = jnp.dot(x_val, rw_ref[:, :], preferred_element_type=jnp.float32)
        s_max = jnp.max(scores, axis=-1, keepdims=True)
        p = jnp.exp(scores - s_max)
        probs = p / jnp.sum(p, axis=-1, keepdims=True)

        route = idx_ref[:, :]
        n_experts = probs.shape[-1]
        eids = lax.broadcasted_iota(jnp.int32, (N_TOK, n_experts), 1)

        ewb_ref[...] = ew_ref[...].astype(jnp.bfloat16)

        acc = jnp.zeros((N_TOK, D_FF), jnp.float32)
        for le in range(E_PER):
            e = my_i * E_PER + le
            prob_e = jnp.sum(
                jnp.where(eids == e, probs, 0.0), axis=-1, keepdims=True
            )
            gate = jnp.where(route == e, prob_e, 0.0)
            acc = acc + jnp.dot(
                (x_val * gate).astype(jnp.bfloat16),
                ewb_ref[le],
                preferred_element_type=jnp.float32,
            )
        acc_ref[:, :] = acc

        for s in range(N_DEV - 1):
            rdmas = []
            for d in range(2):
                if d == 0:
                    off = ((my_i - s) % N_DEV) * CHUNK
                else:
                    off = ((my_i + s) % N_DEV) * CHUNK + HALF
                rs_send[d, s] = acc_ref[pl.ds(off, HALF), :].astype(jnp.bfloat16)
                rdma = pltpu.make_async_remote_copy(
                    src_ref=rs_send.at[d, s],
                    dst_ref=rs_recv.at[d, s],
                    send_sem=rs_ssem.at[d, s],
                    recv_sem=rs_rsem.at[d, s],
                    device_id=(nbr[d],),
                    device_id_type=pl.DeviceIdType.MESH,
                )
                rdma.start()
                rdmas.append(rdma)
            for d in range(2):
                rdmas[d].wait()
                if d == 0:
                    toff = ((my_i - 1 - s) % N_DEV) * CHUNK
                else:
                    toff = ((my_i + 1 + s) % N_DEV) * CHUNK + HALF
                acc_ref[pl.ds(toff, HALF), :] = (
                    acc_ref[pl.ds(toff, HALF), :]
                    + rs_recv[d, s].astype(jnp.float32)
                )

        swb = sw_ref[:, :].astype(jnp.bfloat16)
        own_off = [((my_i + 1) % N_DEV) * CHUNK,
                   ((my_i - 1) % N_DEV) * CHUNK + HALF]
        for d in range(2):
            out_ref[pl.ds(own_off[d], HALF), :] = (
                acc_ref[pl.ds(own_off[d], HALF), :]
                + jnp.dot(
                    x_ref[pl.ds(own_off[d], HALF), :].astype(jnp.bfloat16),
                    swb,
                    preferred_element_type=jnp.float32,
                )
            )
            ag_send0[d] = out_ref[pl.ds(own_off[d], HALF), :].astype(jnp.bfloat16)

        for s in range(N_DEV - 1):
            rdmas = []
            for d in range(2):
                src = ag_send0.at[d] if s == 0 else ag_recv.at[d, s - 1]
                rdma = pltpu.make_async_remote_copy(
                    src_ref=src,
                    dst_ref=ag_recv.at[d, s],
                    send_sem=ag_ssem.at[d, s],
                    recv_sem=ag_rsem.at[d, s],
                    device_id=(nbr[d],),
                    device_id_type=pl.DeviceIdType.MESH,
                )
                rdma.start()
                rdmas.append(rdma)
            for d in range(2):
                rdmas[d].wait()
                if d == 0:
                    roff = ((my_i - s) % N_DEV) * CHUNK
                else:
                    roff = ((my_i + s) % N_DEV) * CHUNK + HALF
                out_ref[pl.ds(roff, HALF), :] = ag_recv[d, s].astype(jnp.float32)

    return pl.pallas_call(
        body,
        out_shape=jax.ShapeDtypeStruct((N_TOK, D_FF), jnp.float32),
        in_specs=[pl.BlockSpec(memory_space=pltpu.VMEM)] * 5,
        out_specs=pl.BlockSpec(memory_space=pltpu.VMEM),
        scratch_shapes=[
            pltpu.VMEM((N_TOK, D_FF), jnp.float32),
            pltpu.VMEM((E_PER, D_MODEL, D_FF), jnp.bfloat16),
            pltpu.VMEM((2, N_DEV - 1, HALF, D_FF), jnp.bfloat16),
            pltpu.VMEM((2, N_DEV - 1, HALF, D_FF), jnp.bfloat16),
            pltpu.VMEM((2, HALF, D_FF), jnp.bfloat16),
            pltpu.VMEM((2, N_DEV - 1, HALF, D_FF), jnp.bfloat16),
            pltpu.SemaphoreType.DMA((2, N_DEV - 1)),
            pltpu.SemaphoreType.DMA((2, N_DEV - 1)),
            pltpu.SemaphoreType.DMA((2, N_DEV - 1)),
            pltpu.SemaphoreType.DMA((2, N_DEV - 1)),
        ],
        compiler_params=pltpu.CompilerParams(collective_id=0),
    )(x, router_W, route_idx, expert_W, shared_W)


# device time: 44156 ns/iter; 2.2150x vs baseline; 1.0829x over previous
import jax
import jax.numpy as jnp
from jax import lax
from jax.experimental import pallas as pl
from jax.experimental.pallas import tpu as pltpu

N_DEV = 4
N_TOK = 1024
D_MODEL = 512
D_FF = 1024
E_PER = 4
CHUNK = N_TOK // N_DEV
HALF = CHUNK // 2

BF16 = jnp.bfloat16
F32 = jnp.float32


def kernel(x, router_W, route_idx, expert_W, shared_W):
    def body(x_ref, rw_ref, idx_ref, ew_ref, sw_ref, out_ref,
             gates_ref, ewb_ref, rs_send, rs_recv, ag_send0, ag_recv,
             rs_ssem, rs_rsem, ag_ssem, ag_rsem):
        my_i = lax.axis_index("i")
        left = (my_i - 1) % N_DEV
        right = (my_i + 1) % N_DEV
        nbr = [right, left]

        def oc(k):
            return ((my_i + k) % N_DEV) * CHUNK

        barrier_sem = pltpu.get_barrier_semaphore()
        for n in [left, right]:
            pl.semaphore_signal(
                barrier_sem, inc=1,
                device_id=(n,), device_id_type=pl.DeviceIdType.MESH,
            )
        pl.semaphore_wait(barrier_sem, 2)

        x_val = x_ref[:, :]
        scores = jnp.dot(x_val, rw_ref[:, :], preferred_element_type=F32)
        s_max = jnp.max(scores, axis=-1, keepdims=True)
        p = jnp.exp(scores - s_max)
        probs = p / jnp.sum(p, axis=-1, keepdims=True)
        route = idx_ref[:, :]
        eids = lax.broadcasted_iota(jnp.int32, (N_TOK, probs.shape[-1]), 1)
        cols = []
        for le in range(E_PER):
            e = my_i * E_PER + le
            prob_e = jnp.sum(
                jnp.where(eids == e, probs, 0.0), axis=-1, keepdims=True
            )
            cols.append(jnp.where(route == e, prob_e, 0.0))
        gates_ref[:, :] = jnp.concatenate(cols, axis=1)

        ewb_ref[...] = ew_ref[...].astype(BF16)

        def part(off, nr):
            xc = x_ref[pl.ds(off, nr), :]
            gc = gates_ref[pl.ds(off, nr), :]
            s = jnp.zeros((nr, D_FF), F32)
            for le in range(E_PER):
                s = s + jnp.dot(
                    (xc * gc[:, le:le + 1]).astype(BF16),
                    ewb_ref[le],
                    preferred_element_type=F32,
                )
            return s

        def start(phase, d, s, src):
            ssem, rsem = (rs_ssem, rs_rsem) if phase == 0 else (ag_ssem, ag_rsem)
            dst = rs_recv if phase == 0 else ag_recv
            rdma = pltpu.make_async_remote_copy(
                src_ref=src, dst_ref=dst.at[d, s],
                send_sem=ssem.at[d, s], recv_sem=rsem.at[d, s],
                device_id=(nbr[d],), device_id_type=pl.DeviceIdType.MESH,
            )
            rdma.start()
            return rdma

        pending = []

        p_own = part(oc(0), CHUNK)
        rs_send[0, 0] = p_own[:HALF].astype(BF16)
        rs_send[1, 0] = p_own[HALF:].astype(BF16)
        cw = start(0, 0, 0, rs_send.at[0, 0])
        ccw = start(0, 1, 0, rs_send.at[1, 0])
        pending += [cw, ccw]

        a_cw0 = part(oc(-1), HALF)
        a_ccw0 = part(oc(1) + HALF, HALF)
        swb = sw_ref[:, :].astype(BF16)
        sh_top = jnp.dot(
            x_ref[pl.ds(oc(1), HALF), :].astype(BF16), swb,
            preferred_element_type=F32,
        )
        sh_bot = jnp.dot(
            x_ref[pl.ds(oc(-1) + HALF, HALF), :].astype(BF16), swb,
            preferred_element_type=F32,
        )
        p_i2 = part(oc(2), CHUNK)

        cw.wait_recv()
        rs_send[0, 1] = (a_cw0 + rs_recv[0, 0].astype(F32)).astype(BF16)
        cw = start(0, 0, 1, rs_send.at[0, 1])
        ccw.wait_recv()
        rs_send[1, 1] = (a_ccw0 + rs_recv[1, 0].astype(F32)).astype(BF16)
        ccw = start(0, 1, 1, rs_send.at[1, 1])
        pending += [cw, ccw]

        a_cw2 = part(oc(1), HALF)
        a_ccw2 = part(oc(-1) + HALF, HALF)

        cw.wait_recv()
        rs_send[0, 2] = (p_i2[:HALF] + rs_recv[0, 1].astype(F32)).astype(BF16)
        cw = start(0, 0, 2, rs_send.at[0, 2])
        ccw.wait_recv()
        rs_send[1, 2] = (p_i2[HALF:] + rs_recv[1, 1].astype(F32)).astype(BF16)
        ccw = start(0, 1, 2, rs_send.at[1, 2])
        pending += [cw, ccw]

        cw.wait_recv()
        own_top = a_cw2 + sh_top + rs_recv[0, 2].astype(F32)
        out_ref[pl.ds(oc(1), HALF), :] = own_top
        ag_send0[0] = own_top.astype(BF16)
        ag_cw = start(1, 0, 0, ag_send0.at[0])
        ccw.wait_recv()
        own_bot = a_ccw2 + sh_bot + rs_recv[1, 2].astype(F32)
        out_ref[pl.ds(oc(-1) + HALF, HALF), :] = own_bot
        ag_send0[1] = own_bot.astype(BF16)
        ag_ccw = start(1, 1, 0, ag_send0.at[1])
        pending += [ag_cw, ag_ccw]

        for s in range(N_DEV - 1):
            ag_cw.wait_recv()
            if s < N_DEV - 2:
                ag_cw = start(1, 0, s + 1, ag_recv.at[0, s])
                pending.append(ag_cw)
            out_ref[pl.ds(oc(-s), HALF), :] = ag_recv[0, s].astype(F32)
            ag_ccw.wait_recv()
            if s < N_DEV - 2:
                ag_ccw = start(1, 1, s + 1, ag_recv.at[1, s])
                pending.append(ag_ccw)
            out_ref[pl.ds(oc(s) + HALF, HALF), :] = ag_recv[1, s].astype(F32)

        for rdma in pending:
            rdma.wait_send()

    return pl.pallas_call(
        body,
        out_shape=jax.ShapeDtypeStruct((N_TOK, D_FF), F32),
        in_specs=[pl.BlockSpec(memory_space=pltpu.VMEM)] * 5,
        out_specs=pl.BlockSpec(memory_space=pltpu.VMEM),
        scratch_shapes=[
            pltpu.VMEM((N_TOK, E_PER), F32),
            pltpu.VMEM((E_PER, D_MODEL, D_FF), BF16),
            pltpu.VMEM((2, N_DEV - 1, HALF, D_FF), BF16),
            pltpu.VMEM((2, N_DEV - 1, HALF, D_FF), BF16),
            pltpu.VMEM((2, HALF, D_FF), BF16),
            pltpu.VMEM((2, N_DEV - 1, HALF, D_FF), BF16),
            pltpu.SemaphoreType.DMA((2, N_DEV - 1)),
            pltpu.SemaphoreType.DMA((2, N_DEV - 1)),
            pltpu.SemaphoreType.DMA((2, N_DEV - 1)),
            pltpu.SemaphoreType.DMA((2, N_DEV - 1)),
        ],
        compiler_params=pltpu.CompilerParams(collective_id=0),
    )(x, router_W, route_idx, expert_W, shared_W)
